# baseline (speedup 1.0000x reference)
import functools

import jax
import jax.numpy as jnp
from jax import lax
from jax.experimental import pallas as pl
from jax.experimental.pallas import tpu as pltpu
from jax.experimental.pallas import tpu_sc as plsc

USER_NUM = 1000000
EMBED_DIM = 64
BATCH = 16384

_NC = 2
_NS = 16
_NW = _NC * _NS
_LANE = 16
_CCOLS = 256
_NCHUNK = -(-USER_NUM // _CCOLS)
_CPW = 124
_NPAIR = _CPW // 2
_MAXOFF = USER_NUM - 192
_NSLOT = 4
_BROWS = 64
_NGRP = _BROWS // _LANE
_DUMP = BATCH
_IB = "promise_in_bounds"


def _issue(table_t, cbuf, sem, g, slot):
    coff = pl.multiple_of(jnp.minimum(g * _CCOLS, _MAXOFF), 128)
    return pltpu.async_copy(
        table_t.at[:, pl.ds(coff, _CCOLS)], cbuf.at[slot], sem
    )


def _wait_chunk(table_t, cbuf, sem, slot):
    pltpu.make_async_copy(
        table_t.at[:, pl.ds(0, _CCOLS)], cbuf.at[slot], sem
    ).wait()


def _body(table_t, idx_hbm, out_hbm, clist, bucket, offs, cnts,
          cbuf, batch, posb, sem_c, sem_s):
    wid = lax.axis_index("s") * _NC + lax.axis_index("c")
    wlo = wid * _CPW
    whi = jnp.minimum(wlo + _CPW, _NCHUNK)
    clo = wlo * _CCOLS
    chi = jnp.minimum(whi * _CCOLS, USER_NUM)

    pltpu.sync_copy(idx_hbm, clist)

    iota = lax.iota(jnp.int32, _LANE)
    zeros16 = jnp.zeros((_LANE,), jnp.int32)
    dump16 = jnp.full((_LANE,), _DUMP, jnp.int32)

    def scan_in(t, cnt):
        v = clist[pl.ds(t * _LANE, _LANE)]
        m = (v >= clo) & (v < chi)
        dv = v - clo
        ps = plsc.cumsum(m.astype(jnp.int32))
        tgt = cnt + ps - 1
        packed = (dv << 14) | (t * _LANE + iota)
        plsc.store_scatter(clist, [tgt], packed, mask=m)
        return cnt + ps[_LANE - 1]

    cnt = lax.fori_loop(0, BATCH // _LANE, scan_in, jnp.int32(0))
    nvec = (cnt + _LANE - 1) // _LANE

    def bucket_pair(p, cur):
        def scanb(t, pcur):
            pk = clist[pl.ds(t * _LANE, _LANE)]
            valid = (t * _LANE + iota) < cnt
            m = ((pk >> 23) == p) & valid
            ps = plsc.cumsum(m.astype(jnp.int32))
            plsc.store_scatter(bucket, [pcur + ps - 1], pk, mask=m)
            return pcur + ps[_LANE - 1]

        nend = lax.fori_loop(0, nvec, scanb, cur)
        n = nend - cur
        p16 = jnp.full((_LANE,), p, jnp.int32)
        lane0 = iota == 0
        plsc.store_scatter(
            offs, [p16], jnp.full((_LANE,), cur, jnp.int32), mask=lane0
        )
        plsc.store_scatter(
            cnts, [p16], jnp.full((_LANE,), n, jnp.int32), mask=lane0
        )
        return cur + ((n + _LANE - 1) & ~(_LANE - 1))

    lax.fori_loop(0, _NPAIR, bucket_pair, jnp.int32(0))

    for par in range(2):
        for k in range(_NGRP):
            plsc.store_scatter(
                posb, [jnp.full((_LANE,), par, jnp.int32), k * _LANE + iota],
                dump16,
            )

    def drain_scatter():
        pltpu.make_async_copy(
            batch.at[0], out_hbm.at[posb.at[0]], sem_s
        ).wait()

    def process_pair(g0, state):
        prel = (g0 - wlo) >> 1
        p16 = jnp.full((_LANE,), prel, jnp.int32)
        base = plsc.load_gather(offs, [p16])[0]
        npair = plsc.load_gather(cnts, [p16])[0]
        ngrp = (npair + _LANE - 1) >> 4

        def group(j, st):
            bcg, fb, pend = st
            pk = bucket[pl.ds(base + j * _LANE, _LANE)]
            valid = (j * _LANE + iota) < npair
            pos = pk & (BATCH - 1)
            vg = clo + (pk >> 14)
            ck = vg >> 8
            slot16 = jnp.where(valid, ck & (_NSLOT - 1), 0)
            loc16 = jnp.where(
                valid,
                (vg & 255) + jnp.where(ck == _NCHUNK - 1, 128, 0),
                0,
            )
            fb16 = jnp.full((_LANE,), fb, jnp.int32)
            brow = bcg * _LANE + iota
            for d in range(EMBED_DIM):
                d16 = jnp.full((_LANE,), d, jnp.int32)
                piece = plsc.load_gather(cbuf, [slot16, d16, loc16])
                plsc.store_scatter(batch, [fb16, brow, d16], piece)
            plsc.store_scatter(
                posb, [fb16, brow], jnp.where(valid, pos, _DUMP)
            )
            do_flush = bcg + 1 == _NGRP

            @pl.when(do_flush)
            def _():
                @pl.when(pend == 1)
                def _():
                    drain_scatter()

                pltpu.async_copy(
                    batch.at[fb], out_hbm.at[posb.at[fb]], sem_s
                )
                nfb16 = jnp.full((_LANE,), 1 - fb, jnp.int32)
                for k in range(_NGRP):
                    plsc.store_scatter(
                        posb, [nfb16, k * _LANE + iota], dump16
                    )

            bcg2 = jnp.where(do_flush, 0, bcg + 1)
            fb2 = jnp.where(do_flush, 1 - fb, fb)
            pend2 = jnp.where(do_flush, 1, pend)
            return bcg2, fb2, pend2

        return lax.fori_loop(0, ngrp, group, state)

    for s in range(2):

        @pl.when(wlo + s < whi)
        def _():
            _issue(table_t, cbuf, sem_c, wlo + s, s)

    def outer(p2, state):
        for pp in range(2):
            g0 = wlo + (p2 * 2 + pp) * 2
            s0 = (pp * 2) % _NSLOT
            for s in range(2):

                @pl.when(g0 + 2 + s < whi)
                def _():
                    _issue(
                        table_t, cbuf, sem_c, g0 + 2 + s,
                        (s0 + 2 + s) % _NSLOT,
                    )

            for s in range(2):

                @pl.when(g0 + s < whi)
                def _():
                    _wait_chunk(table_t, cbuf, sem_c, s0 + s)

            state = lax.cond(
                g0 < whi,
                lambda st: process_pair(g0, st),
                lambda st: st,
                state,
            )
        return state

    bfin, ffin, pfin = lax.fori_loop(
        0, _CPW // 4, outer, (jnp.int32(0), jnp.int32(0), jnp.int32(0))
    )

    @pl.when(pfin == 1)
    def _():
        drain_scatter()

    @pl.when(bfin > 0)
    def _():
        pltpu.async_copy(
            batch.at[ffin], out_hbm.at[posb.at[ffin]], sem_s
        )
        drain_scatter()


@jax.jit
def kernel(nodes, embedding_weight):
    mesh = plsc.VectorSubcoreMesh(core_axis_name="c", subcore_axis_name="s")
    run = functools.partial(
        pl.kernel,
        mesh=mesh,
        out_type=jax.ShapeDtypeStruct((BATCH + 1, 128), jnp.float32),
        scratch_types=[
            pltpu.VMEM((BATCH,), jnp.int32),
            pltpu.VMEM((BATCH + _NPAIR * _LANE,), jnp.int32),
            pltpu.VMEM((_NPAIR,), jnp.int32),
            pltpu.VMEM((_NPAIR,), jnp.int32),
            pltpu.VMEM((_NSLOT, EMBED_DIM, _CCOLS), jnp.float32),
            pltpu.VMEM((2, _BROWS, 128), jnp.float32),
            pltpu.VMEM((2, _BROWS), jnp.int32),
            pltpu.SemaphoreType.DMA,
            pltpu.SemaphoreType.DMA,
        ],
        compiler_params=pltpu.CompilerParams(
            use_tc_tiling_on_sc=True, needs_layout_passes=False
        ),
    )(_body)
    padded = run(embedding_weight.T, nodes.astype(jnp.int32))
    return padded[:BATCH, :EMBED_DIM]


# speedup vs baseline: 4.8342x; 4.8342x over previous
import functools

import jax
import jax.numpy as jnp
from jax import lax
from jax.experimental import pallas as pl
from jax.experimental.pallas import tpu as pltpu
from jax.experimental.pallas import tpu_sc as plsc

USER_NUM = 1000000
EMBED_DIM = 64
BATCH = 16384

_NC = 2
_NS = 16
_NW = _NC * _NS
_LANE = 16
_CCOLS = 256
_NCHUNK = -(-USER_NUM // _CCOLS)
_CPW = 124
_NPAIR = _CPW // 2
_MAXOFF = USER_NUM - 192
_NSLOT = 4
_BROWS = 64
_NGRP = _BROWS // _LANE
_DUMP = BATCH
_IB = "promise_in_bounds"


def _issue(table_t, cbuf, sem, g, slot):
    coff = pl.multiple_of(jnp.minimum(g * _CCOLS, _MAXOFF), 128)
    return pltpu.async_copy(
        table_t.at[:, pl.ds(coff, _CCOLS)], cbuf.at[slot], sem
    )


def _wait_chunk(table_t, cbuf, sem, slot):
    pltpu.make_async_copy(
        table_t.at[:, pl.ds(0, _CCOLS)], cbuf.at[slot], sem
    ).wait()


def _body(table_t, idx_hbm, out_hbm, clist, bucket, offs, cnts,
          cbuf, batch, posb, sem_c, sem_s):
    wid = lax.axis_index("s") * _NC + lax.axis_index("c")
    wlo = wid * _CPW
    whi = jnp.minimum(wlo + _CPW, _NCHUNK)
    clo = wlo * _CCOLS
    chi = jnp.minimum(whi * _CCOLS, USER_NUM)

    pltpu.sync_copy(idx_hbm, clist)

    iota = lax.iota(jnp.int32, _LANE)
    zeros16 = jnp.zeros((_LANE,), jnp.int32)
    dump16 = jnp.full((_LANE,), _DUMP, jnp.int32)

    def scan_in(t, cnt):
        v = clist[pl.ds(t * _LANE, _LANE)]
        m = (v >= clo) & (v < chi)
        dv = v - clo
        ps = plsc.cumsum(m.astype(jnp.int32))
        tgt = cnt + ps - 1
        packed = (dv << 14) | (t * _LANE + iota)
        plsc.store_scatter(clist, [tgt], packed, mask=m)
        return cnt + ps[_LANE - 1]

    cnt = lax.fori_loop(0, BATCH // _LANE, scan_in, jnp.int32(0))
    nvec = (cnt + _LANE - 1) // _LANE

    def bucket_pair(p, cur):
        def scanb(t, pcur):
            pk = clist[pl.ds(t * _LANE, _LANE)]
            valid = (t * _LANE + iota) < cnt
            m = ((pk >> 23) == p) & valid
            ps = plsc.cumsum(m.astype(jnp.int32))
            plsc.store_scatter(bucket, [pcur + ps - 1], pk, mask=m)
            return pcur + ps[_LANE - 1]

        nend = lax.fori_loop(0, nvec, scanb, cur)
        n = nend - cur
        p16 = jnp.full((_LANE,), p, jnp.int32)
        lane0 = iota == 0
        plsc.store_scatter(
            offs, [p16], jnp.full((_LANE,), cur, jnp.int32), mask=lane0
        )
        plsc.store_scatter(
            cnts, [p16], jnp.full((_LANE,), n, jnp.int32), mask=lane0
        )
        return cur + ((n + _LANE - 1) & ~(_LANE - 1))

    lax.fori_loop(0, _NPAIR, bucket_pair, jnp.int32(0))

    for par in range(2):
        for k in range(_NGRP):
            plsc.store_scatter(
                posb, [jnp.full((_LANE,), par, jnp.int32), k * _LANE + iota],
                dump16,
            )

    def drain_scatter():
        pltpu.make_async_copy(
            batch.at[0], out_hbm.at[posb.at[0]], sem_s
        ).wait()

    def process_pair(g0, state):
        prel = (g0 - wlo) >> 1
        p16 = jnp.full((_LANE,), prel, jnp.int32)
        base = plsc.load_gather(offs, [p16])[0]
        npair = plsc.load_gather(cnts, [p16])[0]
        ngrp = (npair + _LANE - 1) >> 4

        def group(jv, st):
            pk = bucket[pl.ds(base + jv * _LANE, _LANE)]
            nin = jnp.minimum(npair - jv * _LANE, _LANE)

            def hit(h, st2):
                bc, fb, pend = st2
                h16 = jnp.full((_LANE,), h, jnp.int32)
                pk16 = pk.at[h16].get(mode=_IB)
                pos16 = pk16 & (BATCH - 1)
                vg = clo + (pk16 >> 14)
                ck16 = vg >> 8
                slot16 = ck16 & (_NSLOT - 1)
                loc16 = (vg & 255) + jnp.where(ck16 == _NCHUNK - 1, 128, 0)
                fb16 = jnp.full((_LANE,), fb, jnp.int32)
                brow = jnp.full((_LANE,), bc, jnp.int32)
                for k in range(EMBED_DIM // _LANE):
                    piece = plsc.load_gather(
                        cbuf, [slot16, iota + k * _LANE, loc16]
                    )
                    plsc.store_scatter(
                        batch, [fb16, brow, k * _LANE + iota], piece
                    )
                plsc.store_scatter(posb, [fb16, brow], pos16, mask=iota == 0)
                do_flush = bc + 1 == _BROWS

                @pl.when(do_flush)
                def _():
                    @pl.when(pend == 1)
                    def _():
                        drain_scatter()

                    pltpu.async_copy(
                        batch.at[fb], out_hbm.at[posb.at[fb]], sem_s
                    )
                    nfb16 = jnp.full((_LANE,), 1 - fb, jnp.int32)
                    for k in range(_NGRP):
                        plsc.store_scatter(
                            posb, [nfb16, k * _LANE + iota], dump16
                        )

                bc2 = jnp.where(do_flush, 0, bc + 1)
                fb2 = jnp.where(do_flush, 1 - fb, fb)
                pend2 = jnp.where(do_flush, 1, pend)
                return bc2, fb2, pend2

            return lax.fori_loop(0, nin, hit, st)

        return lax.fori_loop(0, ngrp, group, state)

    for s in range(2):

        @pl.when(wlo + s < whi)
        def _():
            _issue(table_t, cbuf, sem_c, wlo + s, s)

    def outer(p2, state):
        for pp in range(2):
            g0 = wlo + (p2 * 2 + pp) * 2
            s0 = (pp * 2) % _NSLOT
            for s in range(2):

                @pl.when(g0 + 2 + s < whi)
                def _():
                    _issue(
                        table_t, cbuf, sem_c, g0 + 2 + s,
                        (s0 + 2 + s) % _NSLOT,
                    )

            for s in range(2):

                @pl.when(g0 + s < whi)
                def _():
                    _wait_chunk(table_t, cbuf, sem_c, s0 + s)

            state = lax.cond(
                g0 < whi,
                lambda st: process_pair(g0, st),
                lambda st: st,
                state,
            )
        return state

    bfin, ffin, pfin = lax.fori_loop(
        0, _CPW // 4, outer, (jnp.int32(0), jnp.int32(0), jnp.int32(0))
    )

    @pl.when(pfin == 1)
    def _():
        drain_scatter()

    @pl.when(bfin > 0)
    def _():
        pltpu.async_copy(
            batch.at[ffin], out_hbm.at[posb.at[ffin]], sem_s
        )
        drain_scatter()


@jax.jit
def kernel(nodes, embedding_weight):
    mesh = plsc.VectorSubcoreMesh(core_axis_name="c", subcore_axis_name="s")
    run = functools.partial(
        pl.kernel,
        mesh=mesh,
        out_type=jax.ShapeDtypeStruct((BATCH + 1, 128), jnp.float32),
        scratch_types=[
            pltpu.VMEM((BATCH,), jnp.int32),
            pltpu.VMEM((BATCH + _NPAIR * _LANE,), jnp.int32),
            pltpu.VMEM((_NPAIR,), jnp.int32),
            pltpu.VMEM((_NPAIR,), jnp.int32),
            pltpu.VMEM((_NSLOT, EMBED_DIM, _CCOLS), jnp.float32),
            pltpu.VMEM((2, _BROWS, 128), jnp.float32),
            pltpu.VMEM((2, _BROWS), jnp.int32),
            pltpu.SemaphoreType.DMA,
            pltpu.SemaphoreType.DMA,
        ],
        compiler_params=pltpu.CompilerParams(
            use_tc_tiling_on_sc=True, needs_layout_passes=False
        ),
    )(_body)
    padded = run(embedding_weight.T, nodes.astype(jnp.int32))
    return padded[:BATCH, :EMBED_DIM]
